# Initial kernel scaffold; baseline (speedup 1.0000x reference)
import jax
import jax.numpy as jnp
from jax.experimental import pallas as pl


def kernel(x, edge_index, edge_attr, A1, bnn1, root1, bias1, A2, bnn2, root2, bias2, A3, bnn3, root3, bias3):
    raise NotImplementedError("write your pallas kernel here")


# speedup vs baseline: 2.6547x; 2.6547x over previous
import functools

import numpy as np
import jax
import jax.numpy as jnp
from jax import lax
from jax.experimental import pallas as pl
from jax.experimental.pallas import tpu as pltpu
from jax.experimental.pallas import tpu_sc as plsc

N = 10000
E = 160000
D = 32
DD = D * D

NC = 2
NS = 16
NW = NC * NS
EPW = E // NW
CH = 40
NCH = EPW // CH
OWN = 624
REM_OFF = NS * OWN
REM = N - REM_OFF

_R_np = np.repeat(np.eye(D, dtype=np.float32), D, axis=1)
_S_np = np.tile(np.eye(D, dtype=np.float32), (D, 1))

BE = 2000
BN = 2000


def _msg_body(ea_ref, xs_ref, a_ref, bnn_ref, r_ref, s_ref, out_ref):
    w = jnp.maximum(
        jnp.dot(ea_ref[...], a_ref[...], preferred_element_type=jnp.float32)
        + bnn_ref[...], 0.0)
    xsrep = jnp.dot(xs_ref[...], r_ref[...], preferred_element_type=jnp.float32)
    out_ref[...] = jnp.dot(w * xsrep, s_ref[...],
                           preferred_element_type=jnp.float32)


def _tc_messages(ea, xs, a, bnn2d, r_c, s_c):
    return pl.pallas_call(
        _msg_body,
        grid=(E // BE,),
        in_specs=[
            pl.BlockSpec((BE, D), lambda i: (i, 0)),
            pl.BlockSpec((BE, D), lambda i: (i, 0)),
            pl.BlockSpec((D, DD), lambda i: (0, 0)),
            pl.BlockSpec((1, DD), lambda i: (0, 0)),
            pl.BlockSpec((D, DD), lambda i: (0, 0)),
            pl.BlockSpec((DD, D), lambda i: (0, 0)),
        ],
        out_specs=pl.BlockSpec((BE, D), lambda i: (i, 0)),
        out_shape=jax.ShapeDtypeStruct((E, D), jnp.float32),
    )(ea, xs, a, bnn2d, r_c, s_c)


def _finish1_body(aggp_ref, cntp_ref, x_ref, root_ref, bias_ref,
                  out_ref, cnt_ref):
    cnt = cntp_ref[0] + cntp_ref[1]
    cnt_ref[...] = cnt
    agg = aggp_ref[0] + aggp_ref[1]
    out_ref[...] = (agg / jnp.maximum(cnt, 1.0)
                    + jnp.dot(x_ref[...], root_ref[...],
                              preferred_element_type=jnp.float32)
                    + bias_ref[...])


def _finish_body(aggp_ref, cnt_ref, x_ref, root_ref, bias_ref, out_ref):
    agg = aggp_ref[0] + aggp_ref[1]
    out_ref[...] = (agg / jnp.maximum(cnt_ref[...], 1.0)
                    + jnp.dot(x_ref[...], root_ref[...],
                              preferred_element_type=jnp.float32)
                    + bias_ref[...])


def _tc_finish1(aggp, cntp, x, root, bias2d):
    return pl.pallas_call(
        _finish1_body,
        grid=(N // BN,),
        in_specs=[
            pl.BlockSpec((NC, BN, D), lambda i: (0, i, 0)),
            pl.BlockSpec((NC, BN, D), lambda i: (0, i, 0)),
            pl.BlockSpec((BN, D), lambda i: (i, 0)),
            pl.BlockSpec((D, D), lambda i: (0, 0)),
            pl.BlockSpec((1, D), lambda i: (0, 0)),
        ],
        out_specs=[
            pl.BlockSpec((BN, D), lambda i: (i, 0)),
            pl.BlockSpec((BN, D), lambda i: (i, 0)),
        ],
        out_shape=[
            jax.ShapeDtypeStruct((N, D), jnp.float32),
            jax.ShapeDtypeStruct((N, D), jnp.float32),
        ],
    )(aggp, cntp, x, root, bias2d)


def _tc_finish(aggp, cntm, x, root, bias2d):
    return pl.pallas_call(
        _finish_body,
        grid=(N // BN,),
        in_specs=[
            pl.BlockSpec((NC, BN, D), lambda i: (0, i, 0)),
            pl.BlockSpec((BN, D), lambda i: (i, 0)),
            pl.BlockSpec((BN, D), lambda i: (i, 0)),
            pl.BlockSpec((D, D), lambda i: (0, 0)),
            pl.BlockSpec((1, D), lambda i: (0, 0)),
        ],
        out_specs=pl.BlockSpec((BN, D), lambda i: (i, 0)),
        out_shape=jax.ShapeDtypeStruct((N, D), jnp.float32),
    )(aggp, cntm, x, root, bias2d)


@functools.lru_cache(maxsize=1)
def _sc_mesh():
    return plsc.VectorSubcoreMesh(core_axis_name="c", subcore_axis_name="s",
                                  num_cores=NC, num_subcores=NS)


def _sc_gather(h, src3):
    @functools.partial(
        pl.kernel,
        out_type=jax.ShapeDtypeStruct((E, D), jnp.float32),
        mesh=_sc_mesh(),
        scratch_types=[
            pltpu.VMEM((NCH, CH), jnp.int32),
            pltpu.VMEM((CH, D), jnp.float32),
            pltpu.VMEM_SHARED((N, D), jnp.float32),
            pltpu.SemaphoreType.DMA,
        ],
    )
    def k(h_hbm, src_hbm, out_hbm, idx_v, rows_v, h_sh, sem):
        cid = lax.axis_index("c")
        sid = lax.axis_index("s")
        wid = sid * NC + cid
        pltpu.sync_copy(h_hbm.at[pl.ds(sid * OWN, OWN)],
                        h_sh.at[pl.ds(sid * OWN, OWN)])

        @pl.when(sid == 0)
        def _():
            pltpu.sync_copy(h_hbm.at[pl.ds(REM_OFF, REM)],
                            h_sh.at[pl.ds(REM_OFF, REM)])

        pltpu.sync_copy(src_hbm.at[wid], idx_v)
        plsc.subcore_barrier()

        def body(j, carry):
            pltpu.async_copy(h_sh.at[idx_v.at[j]], rows_v, sem).wait()
            pltpu.sync_copy(rows_v, out_hbm.at[pl.ds(wid * EPW + j * CH, CH)])
            return carry

        lax.fori_loop(0, NCH, body, 0)

    return k(h, src3)


def _sc_scatter_cnt(msg, dst3, zeros_nd, ones_cd):
    @functools.partial(
        pl.kernel,
        out_type=(
            jax.ShapeDtypeStruct((NC, N, D), jnp.float32),
            jax.ShapeDtypeStruct((NC, N, D), jnp.float32),
        ),
        mesh=_sc_mesh(),
        scratch_types=[
            pltpu.VMEM((NCH, CH), jnp.int32),
            pltpu.VMEM((CH, D), jnp.float32),
            pltpu.VMEM((CH, D), jnp.float32),
            pltpu.VMEM_SHARED((N, D), jnp.float32),
            pltpu.VMEM_SHARED((N, D), jnp.float32),
        ],
    )
    def k(msg_hbm, dst_hbm, z_hbm, ones_hbm, agg_out, cnt_out,
          idx_v, rows_v, ones_v, agg_sh, cnt_sh):
        cid = lax.axis_index("c")
        sid = lax.axis_index("s")
        wid = sid * NC + cid
        pltpu.sync_copy(z_hbm.at[pl.ds(sid * OWN, OWN)],
                        agg_sh.at[pl.ds(sid * OWN, OWN)])
        pltpu.sync_copy(z_hbm.at[pl.ds(sid * OWN, OWN)],
                        cnt_sh.at[pl.ds(sid * OWN, OWN)])

        @pl.when(sid == 0)
        def _():
            pltpu.sync_copy(z_hbm.at[pl.ds(REM_OFF, REM)],
                            agg_sh.at[pl.ds(REM_OFF, REM)])
            pltpu.sync_copy(z_hbm.at[pl.ds(REM_OFF, REM)],
                            cnt_sh.at[pl.ds(REM_OFF, REM)])

        pltpu.sync_copy(ones_hbm, ones_v)
        pltpu.sync_copy(dst_hbm.at[wid], idx_v)
        plsc.subcore_barrier()

        def body(j, carry):
            pltpu.sync_copy(msg_hbm.at[pl.ds(wid * EPW + j * CH, CH)], rows_v)
            pltpu.sync_copy(rows_v, agg_sh.at[idx_v.at[j]], add=True)
            pltpu.sync_copy(ones_v, cnt_sh.at[idx_v.at[j]], add=True)
            return carry

        lax.fori_loop(0, NCH, body, 0)
        plsc.subcore_barrier()
        pltpu.sync_copy(agg_sh.at[pl.ds(sid * OWN, OWN)],
                        agg_out.at[cid, pl.ds(sid * OWN, OWN)])
        pltpu.sync_copy(cnt_sh.at[pl.ds(sid * OWN, OWN)],
                        cnt_out.at[cid, pl.ds(sid * OWN, OWN)])

        @pl.when(sid == 0)
        def _():
            pltpu.sync_copy(agg_sh.at[pl.ds(REM_OFF, REM)],
                            agg_out.at[cid, pl.ds(REM_OFF, REM)])
            pltpu.sync_copy(cnt_sh.at[pl.ds(REM_OFF, REM)],
                            cnt_out.at[cid, pl.ds(REM_OFF, REM)])

    return k(msg, dst3, zeros_nd, ones_cd)


def _sc_scatter(msg, dst3, zeros_nd):
    @functools.partial(
        pl.kernel,
        out_type=jax.ShapeDtypeStruct((NC, N, D), jnp.float32),
        mesh=_sc_mesh(),
        scratch_types=[
            pltpu.VMEM((NCH, CH), jnp.int32),
            pltpu.VMEM((CH, D), jnp.float32),
            pltpu.VMEM_SHARED((N, D), jnp.float32),
        ],
    )
    def k(msg_hbm, dst_hbm, z_hbm, agg_out, idx_v, rows_v, agg_sh):
        cid = lax.axis_index("c")
        sid = lax.axis_index("s")
        wid = sid * NC + cid
        pltpu.sync_copy(z_hbm.at[pl.ds(sid * OWN, OWN)],
                        agg_sh.at[pl.ds(sid * OWN, OWN)])

        @pl.when(sid == 0)
        def _():
            pltpu.sync_copy(z_hbm.at[pl.ds(REM_OFF, REM)],
                            agg_sh.at[pl.ds(REM_OFF, REM)])

        pltpu.sync_copy(dst_hbm.at[wid], idx_v)
        plsc.subcore_barrier()

        def body(j, carry):
            pltpu.sync_copy(msg_hbm.at[pl.ds(wid * EPW + j * CH, CH)], rows_v)
            pltpu.sync_copy(rows_v, agg_sh.at[idx_v.at[j]], add=True)
            return carry

        lax.fori_loop(0, NCH, body, 0)
        plsc.subcore_barrier()
        pltpu.sync_copy(agg_sh.at[pl.ds(sid * OWN, OWN)],
                        agg_out.at[cid, pl.ds(sid * OWN, OWN)])

        @pl.when(sid == 0)
        def _():
            pltpu.sync_copy(agg_sh.at[pl.ds(REM_OFF, REM)],
                            agg_out.at[cid, pl.ds(REM_OFF, REM)])

    return k(msg, dst3, zeros_nd)


def kernel(x, edge_index, edge_attr, A1, bnn1, root1, bias1,
           A2, bnn2, root2, bias2, A3, bnn3, root3, bias3):
    src3 = edge_index[0].reshape(NW, NCH, CH)
    dst3 = edge_index[1].reshape(NW, NCH, CH)
    r_c = jnp.asarray(_R_np)
    s_c = jnp.asarray(_S_np)
    zeros_nd = jnp.zeros((N, D), jnp.float32)
    ones_cd = jnp.ones((CH, D), jnp.float32)

    h = x
    cntm = None
    for li, (a, bnn, root, bias) in enumerate((
            (A1, bnn1, root1, bias1),
            (A2, bnn2, root2, bias2),
            (A3, bnn3, root3, bias3))):
        xs = _sc_gather(h, src3)
        msg = _tc_messages(edge_attr, xs, a, bnn.reshape(1, DD), r_c, s_c)
        if li == 0:
            aggp, cntp = _sc_scatter_cnt(msg, dst3, zeros_nd, ones_cd)
            h, cntm = _tc_finish1(aggp, cntp, h, root, bias.reshape(1, D))
        else:
            aggp = _sc_scatter(msg, dst3, zeros_nd)
            h = _tc_finish(aggp, cntm, h, root, bias.reshape(1, D))
    return h


# speedup vs baseline: 2.6965x; 1.0157x over previous
import functools

import numpy as np
import jax
import jax.numpy as jnp
from jax import lax
from jax.experimental import pallas as pl
from jax.experimental.pallas import tpu as pltpu
from jax.experimental.pallas import tpu_sc as plsc

N = 10000
E = 160000
D = 32
DD = D * D

NC = 2
NS = 16
NW = NC * NS
EPW = E // NW
CH = 40
NCH = EPW // CH
GRP = 5
NGRP = NCH // GRP
GROWS = GRP * CH
OWN = 624
REM_OFF = NS * OWN
REM = N - REM_OFF

_R_np = np.repeat(np.eye(D, dtype=np.float32), D, axis=1)
_S_np = np.tile(np.eye(D, dtype=np.float32), (D, 1))

BE = 2000
BN = 2000


def _msg_body(ea_ref, xs_ref, a_ref, bnn_ref, r_ref, s_ref, out_ref):
    w = jnp.maximum(
        jnp.dot(ea_ref[...], a_ref[...], preferred_element_type=jnp.float32)
        + bnn_ref[...], 0.0)
    xsrep = jnp.dot(xs_ref[...], r_ref[...], preferred_element_type=jnp.float32)
    out_ref[...] = jnp.dot(w * xsrep, s_ref[...],
                           preferred_element_type=jnp.float32)


def _tc_messages(ea, xs, a, bnn2d, r_c, s_c):
    return pl.pallas_call(
        _msg_body,
        grid=(E // BE,),
        in_specs=[
            pl.BlockSpec((BE, D), lambda i: (i, 0)),
            pl.BlockSpec((BE, D), lambda i: (i, 0)),
            pl.BlockSpec((D, DD), lambda i: (0, 0)),
            pl.BlockSpec((1, DD), lambda i: (0, 0)),
            pl.BlockSpec((D, DD), lambda i: (0, 0)),
            pl.BlockSpec((DD, D), lambda i: (0, 0)),
        ],
        out_specs=pl.BlockSpec((BE, D), lambda i: (i, 0)),
        out_shape=jax.ShapeDtypeStruct((E, D), jnp.float32),
    )(ea, xs, a, bnn2d, r_c, s_c)


def _finish1_body(aggp_ref, cntp_ref, x_ref, root_ref, bias_ref,
                  out_ref, cnt_ref):
    cnt = cntp_ref[0] + cntp_ref[1]
    cnt_ref[...] = cnt
    agg = aggp_ref[0] + aggp_ref[1]
    out_ref[...] = (agg / jnp.maximum(cnt, 1.0)
                    + jnp.dot(x_ref[...], root_ref[...],
                              preferred_element_type=jnp.float32)
                    + bias_ref[...])


def _finish_body(aggp_ref, cnt_ref, x_ref, root_ref, bias_ref, out_ref):
    agg = aggp_ref[0] + aggp_ref[1]
    out_ref[...] = (agg / jnp.maximum(cnt_ref[...], 1.0)
                    + jnp.dot(x_ref[...], root_ref[...],
                              preferred_element_type=jnp.float32)
                    + bias_ref[...])


def _tc_finish1(aggp, cntp, x, root, bias2d):
    return pl.pallas_call(
        _finish1_body,
        grid=(N // BN,),
        in_specs=[
            pl.BlockSpec((NC, BN, D), lambda i: (0, i, 0)),
            pl.BlockSpec((NC, BN, D), lambda i: (0, i, 0)),
            pl.BlockSpec((BN, D), lambda i: (i, 0)),
            pl.BlockSpec((D, D), lambda i: (0, 0)),
            pl.BlockSpec((1, D), lambda i: (0, 0)),
        ],
        out_specs=[
            pl.BlockSpec((BN, D), lambda i: (i, 0)),
            pl.BlockSpec((BN, D), lambda i: (i, 0)),
        ],
        out_shape=[
            jax.ShapeDtypeStruct((N, D), jnp.float32),
            jax.ShapeDtypeStruct((N, D), jnp.float32),
        ],
    )(aggp, cntp, x, root, bias2d)


def _tc_finish(aggp, cntm, x, root, bias2d):
    return pl.pallas_call(
        _finish_body,
        grid=(N // BN,),
        in_specs=[
            pl.BlockSpec((NC, BN, D), lambda i: (0, i, 0)),
            pl.BlockSpec((BN, D), lambda i: (i, 0)),
            pl.BlockSpec((BN, D), lambda i: (i, 0)),
            pl.BlockSpec((D, D), lambda i: (0, 0)),
            pl.BlockSpec((1, D), lambda i: (0, 0)),
        ],
        out_specs=pl.BlockSpec((BN, D), lambda i: (i, 0)),
        out_shape=jax.ShapeDtypeStruct((N, D), jnp.float32),
    )(aggp, cntm, x, root, bias2d)


@functools.lru_cache(maxsize=1)
def _sc_mesh():
    return plsc.VectorSubcoreMesh(core_axis_name="c", subcore_axis_name="s",
                                  num_cores=NC, num_subcores=NS)


def _sc_gather(h, src3):
    @functools.partial(
        pl.kernel,
        out_type=jax.ShapeDtypeStruct((E, D), jnp.float32),
        mesh=_sc_mesh(),
        scratch_types=[
            pltpu.VMEM((NCH, CH), jnp.int32),
            [pltpu.VMEM((CH, D), jnp.float32)] * GRP,
            pltpu.VMEM_SHARED((N, D), jnp.float32),
            [pltpu.SemaphoreType.DMA] * GRP,
            pltpu.SemaphoreType.DMA,
        ],
    )
    def k(h_hbm, src_hbm, out_hbm, idx_v, bufs, h_sh, gsems, wsem):
        cid = lax.axis_index("c")
        sid = lax.axis_index("s")
        wid = sid * NC + cid
        pltpu.sync_copy(h_hbm.at[pl.ds(sid * OWN, OWN)],
                        h_sh.at[pl.ds(sid * OWN, OWN)])

        @pl.when(sid == 0)
        def _():
            pltpu.sync_copy(h_hbm.at[pl.ds(REM_OFF, REM)],
                            h_sh.at[pl.ds(REM_OFF, REM)])

        pltpu.sync_copy(src_hbm.at[wid], idx_v)
        plsc.subcore_barrier()

        def body(g, carry):
            cps = [pltpu.async_copy(h_sh.at[idx_v.at[g * GRP + b]],
                                    bufs[b], gsems[b])
                   for b in range(GRP)]
            for b in range(GRP):
                cps[b].wait()
                pltpu.sync_copy(
                    bufs[b],
                    out_hbm.at[pl.ds(wid * EPW + (g * GRP + b) * CH, CH)])
            return carry

        lax.fori_loop(0, NGRP, body, 0)

    return k(h, src3)


def _sc_scatter_cnt(msg, dst3, zeros_nd, ones_cd):
    @functools.partial(
        pl.kernel,
        out_type=(
            jax.ShapeDtypeStruct((NC, N, D), jnp.float32),
            jax.ShapeDtypeStruct((NC, N, D), jnp.float32),
        ),
        mesh=_sc_mesh(),
        scratch_types=[
            pltpu.VMEM((NCH, CH), jnp.int32),
            pltpu.VMEM((CH, D), jnp.float32),
            pltpu.VMEM((CH, D), jnp.float32),
            pltpu.VMEM_SHARED((N, D), jnp.float32),
            pltpu.VMEM_SHARED((N, D), jnp.float32),
        ],
    )
    def k(msg_hbm, dst_hbm, z_hbm, ones_hbm, agg_out, cnt_out,
          idx_v, rows_v, ones_v, agg_sh, cnt_sh):
        cid = lax.axis_index("c")
        sid = lax.axis_index("s")
        wid = sid * NC + cid
        pltpu.sync_copy(z_hbm.at[pl.ds(sid * OWN, OWN)],
                        agg_sh.at[pl.ds(sid * OWN, OWN)])
        pltpu.sync_copy(z_hbm.at[pl.ds(sid * OWN, OWN)],
                        cnt_sh.at[pl.ds(sid * OWN, OWN)])

        @pl.when(sid == 0)
        def _():
            pltpu.sync_copy(z_hbm.at[pl.ds(REM_OFF, REM)],
                            agg_sh.at[pl.ds(REM_OFF, REM)])
            pltpu.sync_copy(z_hbm.at[pl.ds(REM_OFF, REM)],
                            cnt_sh.at[pl.ds(REM_OFF, REM)])

        pltpu.sync_copy(ones_hbm, ones_v)
        pltpu.sync_copy(dst_hbm.at[wid], idx_v)
        plsc.subcore_barrier()

        def body(j, carry):
            pltpu.sync_copy(msg_hbm.at[pl.ds(wid * EPW + j * CH, CH)], rows_v)
            pltpu.sync_copy(rows_v, agg_sh.at[idx_v.at[j]], add=True)
            pltpu.sync_copy(ones_v, cnt_sh.at[idx_v.at[j]], add=True)
            return carry

        lax.fori_loop(0, NCH, body, 0)
        plsc.subcore_barrier()
        pltpu.sync_copy(agg_sh.at[pl.ds(sid * OWN, OWN)],
                        agg_out.at[cid, pl.ds(sid * OWN, OWN)])
        pltpu.sync_copy(cnt_sh.at[pl.ds(sid * OWN, OWN)],
                        cnt_out.at[cid, pl.ds(sid * OWN, OWN)])

        @pl.when(sid == 0)
        def _():
            pltpu.sync_copy(agg_sh.at[pl.ds(REM_OFF, REM)],
                            agg_out.at[cid, pl.ds(REM_OFF, REM)])
            pltpu.sync_copy(cnt_sh.at[pl.ds(REM_OFF, REM)],
                            cnt_out.at[cid, pl.ds(REM_OFF, REM)])

    return k(msg, dst3, zeros_nd, ones_cd)


def _sc_scatter(msg, dst3, zeros_nd):
    @functools.partial(
        pl.kernel,
        out_type=jax.ShapeDtypeStruct((NC, N, D), jnp.float32),
        mesh=_sc_mesh(),
        scratch_types=[
            pltpu.VMEM((NCH, CH), jnp.int32),
            pltpu.VMEM((CH, D), jnp.float32),
            pltpu.VMEM_SHARED((N, D), jnp.float32),
        ],
    )
    def k(msg_hbm, dst_hbm, z_hbm, agg_out, idx_v, rows_v, agg_sh):
        cid = lax.axis_index("c")
        sid = lax.axis_index("s")
        wid = sid * NC + cid
        pltpu.sync_copy(z_hbm.at[pl.ds(sid * OWN, OWN)],
                        agg_sh.at[pl.ds(sid * OWN, OWN)])

        @pl.when(sid == 0)
        def _():
            pltpu.sync_copy(z_hbm.at[pl.ds(REM_OFF, REM)],
                            agg_sh.at[pl.ds(REM_OFF, REM)])

        pltpu.sync_copy(dst_hbm.at[wid], idx_v)
        plsc.subcore_barrier()

        def body(j, carry):
            pltpu.sync_copy(msg_hbm.at[pl.ds(wid * EPW + j * CH, CH)], rows_v)
            pltpu.sync_copy(rows_v, agg_sh.at[idx_v.at[j]], add=True)
            return carry

        lax.fori_loop(0, NCH, body, 0)
        plsc.subcore_barrier()
        pltpu.sync_copy(agg_sh.at[pl.ds(sid * OWN, OWN)],
                        agg_out.at[cid, pl.ds(sid * OWN, OWN)])

        @pl.when(sid == 0)
        def _():
            pltpu.sync_copy(agg_sh.at[pl.ds(REM_OFF, REM)],
                            agg_out.at[cid, pl.ds(REM_OFF, REM)])

    return k(msg, dst3, zeros_nd)


def kernel(x, edge_index, edge_attr, A1, bnn1, root1, bias1,
           A2, bnn2, root2, bias2, A3, bnn3, root3, bias3):
    src3 = edge_index[0].reshape(NW, NCH, CH)
    dst3 = edge_index[1].reshape(NW, NCH, CH)
    r_c = jnp.asarray(_R_np)
    s_c = jnp.asarray(_S_np)
    zeros_nd = jnp.zeros((N, D), jnp.float32)
    ones_cd = jnp.ones((CH, D), jnp.float32)

    h = x
    cntm = None
    for li, (a, bnn, root, bias) in enumerate((
            (A1, bnn1, root1, bias1),
            (A2, bnn2, root2, bias2),
            (A3, bnn3, root3, bias3))):
        xs = _sc_gather(h, src3)
        msg = _tc_messages(edge_attr, xs, a, bnn.reshape(1, DD), r_c, s_c)
        if li == 0:
            aggp, cntp = _sc_scatter_cnt(msg, dst3, zeros_nd, ones_cd)
            h, cntm = _tc_finish1(aggp, cntp, h, root, bias.reshape(1, D))
        else:
            aggp = _sc_scatter(msg, dst3, zeros_nd)
            h = _tc_finish(aggp, cntm, h, root, bias.reshape(1, D))
    return h
